# baseline (speedup 1.0000x reference)
import functools

import jax
import jax.numpy as jnp
from jax import lax
from jax.experimental import pallas as pl
from jax.experimental.pallas import tpu as pltpu
from jax.experimental.pallas import tpu_sc as plsc

N = 10000
N2 = 20000
N2P = 20480
NLOW = 10112
PLOW = 10240
H = 128
E = 320000
NT = 16
NW = 32
CE = 64
NCH = 159
EPT = NCH * CE
EPAD = NW * EPT
DT = 12.0 / 2.0
RB = 2048
GRID = N2P // RB
LOWB = PLOW // RB

_f32 = jnp.float32

_GDN = lax.GatherDimensionNumbers(
    offset_dims=(), collapsed_slice_dims=(0,), start_index_map=(0,))


def _bcast_lane(vec, e):
    idx = jnp.full((16, 1), e, jnp.int32)
    return lax.gather(vec, idx, _GDN, slice_sizes=(1,),
                      mode=lax.GatherScatterMode.PROMISE_IN_BOUNDS)


def _spmm_call(T, rc3, w3):
    mesh = plsc.VectorSubcoreMesh(core_axis_name="c", subcore_axis_name="s")

    @functools.partial(
        pl.kernel,
        mesh=mesh,
        out_type=jax.ShapeDtypeStruct((2, PLOW, H), _f32),
        scratch_types=[
            pltpu.VMEM((EPT,), jnp.int32),
            pltpu.VMEM((EPT,), _f32),
            pltpu.VMEM((3, CE), jnp.int32),
            pltpu.VMEM((3, CE), jnp.int32),
            pltpu.VMEM((CE, H), _f32),
            pltpu.VMEM((CE, H), _f32),
            pltpu.VMEM((CE, H), _f32),
            pltpu.VMEM_SHARED((NLOW, H), _f32),
            pltpu.SemaphoreType.DMA,
            pltpu.SemaphoreType.DMA,
            pltpu.SemaphoreType.DMA,
            pltpu.SemaphoreType.DMA,
            pltpu.SemaphoreType.DMA,
            pltpu.SemaphoreType.DMA,
        ],
    )
    def spmm(t_hbm, rc_hbm, w_hbm, out_hbm,
             rc_v, w_v, rowc, colc, rows0, rows1, rows2, acc,
             gsem0, gsem1, gsem2, ssem0, ssem1, ssem2):
        c = lax.axis_index("c")
        s = lax.axis_index("s")
        eslice = c * NT + s
        nrt = NLOW // NT
        sl = pl.ds(s * nrt, nrt)
        bufs = (rows0, rows1, rows2)
        gsems = (gsem0, gsem1, gsem2)
        ssems = (ssem0, ssem1, ssem2)

        pltpu.sync_copy(rc_hbm.at[eslice], rc_v)
        pltpu.sync_copy(w_hbm.at[eslice], w_v)

        z = jnp.zeros((16,), _f32)
        for r in range(CE):
            rr = rows0.at[r]
            for fg in range(H // 16):
                rr[pl.ds(fg * 16, 16)] = z
        for q in range(nrt // CE):
            pltpu.sync_copy(rows0, acc.at[pl.ds(s * nrt + q * CE, CE)])
        rem = nrt - (nrt // CE) * CE
        if rem:
            pltpu.sync_copy(rows0.at[pl.ds(0, rem)],
                            acc.at[pl.ds(s * nrt + nrt - rem, rem)])

        mask16 = jnp.full((16,), 0xFFFF, jnp.int32)

        def unpack(j, b):
            for k in range(CE // 16):
                pv = rc_v[pl.ds(j * CE + k * 16, 16)]
                rowc.at[b][pl.ds(k * 16, 16)] = pv & mask16
                colc.at[b][pl.ds(k * 16, 16)] = pv >> 16

        for b in range(3):
            unpack(b, b)
            pltpu.async_copy(t_hbm.at[rowc.at[b]], bufs[b], gsems[b])
        plsc.subcore_barrier()

        def scale(buf, j):
            def sk(k, carry):
                wvec = w_v[pl.ds(j * CE + k * 16, 16)]
                for e in range(16):
                    wsp = _bcast_lane(wvec, e)
                    rr = buf.at[k * 16 + e]
                    for fg in range(H // 16):
                        rr[pl.ds(fg * 16, 16)] = (
                            rr[pl.ds(fg * 16, 16)] * wsp)
                return carry

            lax.fori_loop(0, CE // 16, sk, 0)

        ngrp = NCH // 3

        def grp(g, carry):
            for b in range(3):
                j = g * 3 + b
                pltpu.make_async_copy(
                    t_hbm.at[rowc.at[b]], bufs[b], gsems[b]).wait()
                scale(bufs[b], j)
                pltpu.async_copy(bufs[b], acc.at[colc.at[b]],
                                 ssems[b], add=True)

                @pl.when(j + 3 < NCH)
                def _():
                    pltpu.make_async_copy(
                        bufs[b], acc.at[colc.at[b]], ssems[b]).wait()
                    unpack(j + 3, b)
                    pltpu.async_copy(
                        t_hbm.at[rowc.at[b]], bufs[b], gsems[b])
            return carry

        lax.fori_loop(0, ngrp, grp, 0)
        for b in range(3):
            pltpu.make_async_copy(
                bufs[b], acc.at[colc.at[b]], ssems[b]).wait()
        plsc.subcore_barrier()
        pltpu.sync_copy(acc.at[sl], out_hbm.at[c].at[sl])

    return spmm(T, rc3, w3)


def _deg_call(rc3, w3):
    mesh = plsc.VectorSubcoreMesh(core_axis_name="c", subcore_axis_name="s")

    @functools.partial(
        pl.kernel,
        mesh=mesh,
        out_type=jax.ShapeDtypeStruct((2, PLOW, H), _f32),
        scratch_types=[
            pltpu.VMEM((EPT,), jnp.int32),
            pltpu.VMEM((EPT,), _f32),
            pltpu.VMEM((1, CE), jnp.int32),
            pltpu.VMEM((CE, H), _f32),
            pltpu.VMEM_SHARED((NLOW, H), _f32),
        ],
    )
    def degk(rc_hbm, w_hbm, out_hbm, rc_v, w_v, colc, rows, acc):
        c = lax.axis_index("c")
        s = lax.axis_index("s")
        eslice = c * NT + s
        nrt = NLOW // NT
        sl = pl.ds(s * nrt, nrt)

        pltpu.sync_copy(rc_hbm.at[eslice], rc_v)
        pltpu.sync_copy(w_hbm.at[eslice], w_v)

        z = jnp.zeros((16,), _f32)
        for r in range(CE):
            rr = rows.at[r]
            for fg in range(H // 16):
                rr[pl.ds(fg * 16, 16)] = z
        for q in range(nrt // CE):
            pltpu.sync_copy(rows, acc.at[pl.ds(s * nrt + q * CE, CE)])
        rem = nrt - (nrt // CE) * CE
        if rem:
            pltpu.sync_copy(rows.at[pl.ds(0, rem)],
                            acc.at[pl.ds(s * nrt + nrt - rem, rem)])
        plsc.subcore_barrier()

        def chunk(j, carry):
            def sk(k, carry2):
                pv = rc_v[pl.ds(j * CE + k * 16, 16)]
                colc.at[0][pl.ds(k * 16, 16)] = pv >> 16
                wvec = w_v[pl.ds(j * CE + k * 16, 16)]
                for e in range(16):
                    wsp = _bcast_lane(wvec, e)
                    rr = rows.at[k * 16 + e]
                    for fg in range(H // 16):
                        rr[pl.ds(fg * 16, 16)] = wsp
                return carry2

            lax.fori_loop(0, CE // 16, sk, 0)
            pltpu.sync_copy(rows, acc.at[colc.at[0]], add=True)
            return carry

        lax.fori_loop(0, NCH, chunk, 0)
        plsc.subcore_barrier()
        pltpu.sync_copy(acc.at[sl], out_hbm.at[c].at[sl])

    return degk(rc3, w3)


def _w_spec(shape):
    return pl.BlockSpec(shape, lambda i: (0,) * len(shape))


_ROW = pl.BlockSpec((RB, H), lambda i: (i, 0))
_ROW1 = pl.BlockSpec((RB, 1), lambda i: (i, 0))
_PLOW = pl.BlockSpec((2, RB, H), lambda i: (0, jnp.minimum(i, LOWB - 1), 0))


def _dot(a, b):
    return jnp.dot(a, b, preferred_element_type=_f32)


def _combine(p_ref, t_ref):
    gr = (pl.program_id(0) * RB
          + lax.broadcasted_iota(jnp.int32, (RB, H), 0))
    return t_ref[...] + jnp.where(gr < NLOW, p_ref[0] + p_ref[1], 0.0)


def _tc_prep(x, deg, w1, b1, w2, b2, gw):
    def body(x_ref, deg_ref, w1_ref, b1_ref, w2_ref, b2_ref, gw_ref, out_ref):
        h = jnp.maximum(x_ref[...] * w1_ref[...] + b1_ref[...], 0.0)
        h = _dot(h, w2_ref[...]) + b2_ref[...]
        out_ref[...] = lax.rsqrt(deg_ref[...]) * _dot(h, gw_ref[...])

    return pl.pallas_call(
        body,
        grid=(GRID,),
        in_specs=[_ROW1, _ROW1, _w_spec((1, H)), _w_spec((1, H)),
                  _w_spec((H, H)), _w_spec((1, H)), _w_spec((H, H))],
        out_specs=_ROW,
        out_shape=jax.ShapeDtypeStruct((N2P, H), _f32),
    )(x, deg, w1, b1, w2, b2, gw)


def _tc_h0(p, t, deg, gb):
    def body(p_ref, t_ref, deg_ref, gb_ref, out_ref):
        sf = _combine(p_ref, t_ref)
        out_ref[...] = jnp.maximum(
            lax.rsqrt(deg_ref[...]) * sf + gb_ref[...], 0.0)

    return pl.pallas_call(
        body,
        grid=(GRID,),
        in_specs=[_PLOW, _ROW, _ROW1, _w_spec((1, H))],
        out_specs=_ROW,
        out_shape=jax.ShapeDtypeStruct((N2P, H), _f32),
    )(p, t, deg, gb)


def _tc_a(x, deg, mw1, mb1, mw2, mb2, gw1):
    def body(x_ref, deg_ref, mw1_ref, mb1_ref, mw2_ref, mb2_ref, gw1_ref,
             hn_ref, t_ref):
        x = x_ref[...]
        hn = _dot(jnp.tanh(_dot(x, mw1_ref[...]) + mb1_ref[...]),
                  mw2_ref[...]) + mb2_ref[...]
        hn_ref[...] = hn
        t_ref[...] = lax.rsqrt(deg_ref[...]) * _dot(x, gw1_ref[...])

    return pl.pallas_call(
        body,
        grid=(GRID,),
        in_specs=[_ROW, _ROW1, _w_spec((H, H)), _w_spec((1, H)),
                  _w_spec((H, H)), _w_spec((1, H)), _w_spec((H, H))],
        out_specs=[_ROW, _ROW],
        out_shape=[jax.ShapeDtypeStruct((N2P, H), _f32),
                   jax.ShapeDtypeStruct((N2P, H), _f32)],
    )(x, deg, mw1, mb1, mw2, mb2, gw1)


def _tc_b(p, t, deg, g1b, gw2):
    def body(p_ref, t_ref, deg_ref, g1b_ref, gw2_ref, out_ref):
        dinv = lax.rsqrt(deg_ref[...])
        sf = _combine(p_ref, t_ref)
        hg1 = jnp.maximum(dinv * sf + g1b_ref[...], 0.0)
        out_ref[...] = dinv * _dot(hg1, gw2_ref[...])

    return pl.pallas_call(
        body,
        grid=(GRID,),
        in_specs=[_PLOW, _ROW, _ROW1, _w_spec((1, H)), _w_spec((H, H))],
        out_specs=_ROW,
        out_shape=jax.ShapeDtypeStruct((N2P, H), _f32),
    )(p, t, deg, g1b, gw2)


def _tc_c(p, t, deg, g2b, hn, gwa, gwb, gb, hcur, ksum, a_next, wk, last):
    def body(p_ref, t_ref, deg_ref, g2b_ref, hn_ref, gwa_ref, gwb_ref,
             gb_ref, hcur_ref, *rest):
        if ksum is not None:
            ksum_ref = rest[0]
            rest = rest[1:]
        if last:
            (hnew_ref,) = rest
        else:
            xn_ref, ks_ref = rest
        hg = lax.rsqrt(deg_ref[...]) * _combine(p_ref, t_ref) + g2b_ref[...]
        hn = hn_ref[...]
        g = jax.nn.sigmoid(_dot(hg, gwa_ref[...]) + _dot(hn, gwb_ref[...])
                           + gb_ref[...])
        k = g * hg + (1.0 - g) * hn
        if last:
            hnew_ref[...] = hcur_ref[...] + (DT / 6.0) * (ksum_ref[...] + k)
        else:
            xn_ref[...] = hcur_ref[...] + a_next * k
            if ksum is None:
                ks_ref[...] = wk * k
            else:
                ks_ref[...] = ksum_ref[...] + wk * k

    in_specs = [_PLOW, _ROW, _ROW1, _w_spec((1, H)), _ROW, _w_spec((H, H)),
                _w_spec((H, H)), _w_spec((1, H)), _ROW]
    args = [p, t, deg, g2b, hn, gwa, gwb, gb, hcur]
    if ksum is not None:
        in_specs.append(_ROW)
        args.append(ksum)
    if last:
        out_specs = _ROW
        out_shape = jax.ShapeDtypeStruct((N2P, H), _f32)
    else:
        out_specs = [_ROW, _ROW]
        out_shape = [jax.ShapeDtypeStruct((N2P, H), _f32),
                     jax.ShapeDtypeStruct((N2P, H), _f32)]
    return pl.pallas_call(
        body,
        grid=(GRID,),
        in_specs=in_specs,
        out_specs=out_specs,
        out_shape=out_shape,
    )(*args)


def _tc_dec(hcur, w1, b1, w2, b2):
    def body(h_ref, w1_ref, b1_ref, w2_ref, b2_ref, out_ref):
        d = jnp.maximum(_dot(h_ref[...], w1_ref[...]) + b1_ref[...], 0.0)
        out_ref[...] = _dot(d, w2_ref[...]) + b2_ref[...]

    return pl.pallas_call(
        body,
        grid=(GRID,),
        in_specs=[_ROW, _w_spec((H, H)), _w_spec((1, H)),
                  _w_spec((H, 1)), _w_spec((1, 1))],
        out_specs=_ROW1,
        out_shape=jax.ShapeDtypeStruct((N2P, 1), _f32),
    )(hcur, w1, b1, w2, b2)


def kernel(X, edge_index, edge_weight, enc_W1, enc_b1, enc_W2, enc_b2,
           gcn_W, gcn_b, mlp_W1, mlp_b1, mlp_W2, mlp_b2,
           gc1_W, gc1_b, gc2_W, gc2_b, gate_W, gate_b,
           dec_W1, dec_b1, dec_W2, dec_b2):
    row = edge_index[0]
    col = edge_index[1]
    pad = EPAD - E
    packed = (col << 16) | row
    rc3 = jnp.pad(packed, (0, pad)).reshape(NW, EPT)
    w3 = jnp.pad(edge_weight, (0, pad)).reshape(NW, EPT)

    x2d = jnp.pad(X[:, -1, :, :].reshape(N2, 1), ((0, N2P - N2), (0, 0)))

    def r2(b):
        return b.reshape(1, -1)

    gwa = gate_W[:H]
    gwb = gate_W[H:]

    pdeg = _deg_call(rc3, w3)
    deg = jnp.concatenate(
        [1.0 + pdeg[0, :N, 0:1] + pdeg[1, :N, 0:1],
         jnp.ones((N2P - N, 1), _f32)])

    t0 = _tc_prep(x2d, deg, r2(enc_W1[0]), r2(enc_b1), enc_W2, r2(enc_b2),
                  gcn_W)
    p0 = _spmm_call(t0, rc3, w3)
    hcur = _tc_h0(p0, t0, deg, r2(gcn_b))

    for _ in range(2):
        ksum = None
        x = hcur
        for i in range(4):
            hn, t1 = _tc_a(x, deg, mlp_W1, r2(mlp_b1), mlp_W2, r2(mlp_b2),
                           gc1_W)
            p1 = _spmm_call(t1, rc3, w3)
            t2 = _tc_b(p1, t1, deg, r2(gc1_b), gc2_W)
            p2 = _spmm_call(t2, rc3, w3)
            if i < 3:
                a_next = 0.5 * DT if i < 2 else DT
                wk = 1.0 if i == 0 else 2.0
                x, ksum = _tc_c(p2, t2, deg, r2(gc2_b), hn, gwa, gwb,
                                r2(gate_b), hcur, ksum, a_next, wk,
                                last=False)
            else:
                hcur = _tc_c(p2, t2, deg, r2(gc2_b), hn, gwa, gwb,
                             r2(gate_b), hcur, ksum, 0.0, 1.0, last=True)

    out = _tc_dec(hcur, dec_W1, r2(dec_b1), dec_W2, r2(dec_b2))
    return out[:N2].reshape(2, N, 1)[:, None, :, :]


# speedup vs baseline: 1.2951x; 1.2951x over previous
import functools

import jax
import jax.numpy as jnp
from jax import lax
from jax.experimental import pallas as pl
from jax.experimental.pallas import tpu as pltpu
from jax.experimental.pallas import tpu_sc as plsc

N = 10000
N2 = 20000
N2P = 20480
NLOW = 10112
PLOW = 10240
H = 128
E = 320000
NT = 16
NW = 32
CE = 112
NCH = 90
EPT = NCH * CE
EPAD = NW * EPT
DT = 12.0 / 2.0
RB = 2048
GRID = N2P // RB
LOWB = PLOW // RB

_f32 = jnp.float32

_GDN = lax.GatherDimensionNumbers(
    offset_dims=(), collapsed_slice_dims=(0,), start_index_map=(0,))


def _bcast_lane(vec, e):
    idx = jnp.full((16, 1), e, jnp.int32)
    return lax.gather(vec, idx, _GDN, slice_sizes=(1,),
                      mode=lax.GatherScatterMode.PROMISE_IN_BOUNDS)


def _spmm_call(T, rc3, w3):
    mesh = plsc.VectorSubcoreMesh(core_axis_name="c", subcore_axis_name="s")
    nbytes = CE * H * 4

    @functools.partial(
        pl.kernel,
        mesh=mesh,
        out_type=jax.ShapeDtypeStruct((2, PLOW, H), _f32),
        scratch_types=[
            pltpu.VMEM((EPT,), jnp.int32),
            pltpu.VMEM((EPT,), _f32),
            pltpu.VMEM((2, CE), jnp.int32),
            pltpu.VMEM((2, CE), jnp.int32),
            pltpu.VMEM((CE, H), _f32),
            pltpu.VMEM((CE, H), _f32),
            pltpu.VMEM_SHARED((NLOW, H), _f32),
            pltpu.SemaphoreType.DMA,
            pltpu.SemaphoreType.DMA,
            pltpu.SemaphoreType.DMA,
            pltpu.SemaphoreType.DMA,
        ],
    )
    def spmm(t_hbm, rc_hbm, w_hbm, out_hbm,
             rc_v, w_v, rowc, colc, rows0, rows1, acc,
             gsem0, gsem1, ssem0, ssem1):
        c = lax.axis_index("c")
        s = lax.axis_index("s")
        eslice = c * NT + s
        nrt = NLOW // NT
        sl = pl.ds(s * nrt, nrt)
        bufs = (rows0, rows1)
        gsems = (gsem0, gsem1)
        ssems = (ssem0, ssem1)

        pltpu.sync_copy(rc_hbm.at[eslice], rc_v)
        pltpu.sync_copy(w_hbm.at[eslice], w_v)

        z = jnp.zeros((16,), _f32)
        for r in range(CE):
            rr = rows0.at[r]
            for fg in range(H // 16):
                rr[pl.ds(fg * 16, 16)] = z
        for q in range(nrt // CE):
            pltpu.sync_copy(rows0, acc.at[pl.ds(s * nrt + q * CE, CE)])
        rem = nrt - (nrt // CE) * CE
        if rem:
            pltpu.sync_copy(rows0.at[pl.ds(0, rem)],
                            acc.at[pl.ds(s * nrt + nrt - rem, rem)])

        mask16 = jnp.full((16,), 0xFFFF, jnp.int32)

        def unpack(j, b):
            for k in range(CE // 16):
                pv = rc_v[pl.ds(j * CE + k * 16, 16)]
                rowc.at[b][pl.ds(k * 16, 16)] = pv & mask16
                colc.at[b][pl.ds(k * 16, 16)] = pv >> 16

        for b in range(2):
            unpack(b, b)
            pltpu.async_copy(t_hbm.at[rowc.at[b]], bufs[b], gsems[b])
        plsc.subcore_barrier()

        def scale(buf, j):
            for k in range(CE // 16):
                wvec = w_v[pl.ds(j * CE + k * 16, 16)]
                for e in range(16):
                    wsp = _bcast_lane(wvec, e)
                    rr = buf.at[k * 16 + e]
                    for fg in range(H // 16):
                        rr[pl.ds(fg * 16, 16)] = (
                            rr[pl.ds(fg * 16, 16)] * wsp)

        ngrp = NCH // 2

        def grp(g, carry):
            for b in range(2):
                j = g * 2 + b
                pltpu.make_async_copy(
                    t_hbm.at[rowc.at[b]], bufs[b], gsems[b]).wait()
                scale(bufs[b], j)
                pltpu.async_copy(bufs[b], acc.at[colc.at[b]],
                                 ssems[b], add=True)

                @pl.when(j + 2 < NCH)
                def _():
                    pltpu.make_async_copy(
                        bufs[b], acc.at[colc.at[b]], ssems[b]).wait()
                    unpack(j + 2, b)
                    pltpu.async_copy(
                        t_hbm.at[rowc.at[b]], bufs[b], gsems[b])
            return carry

        lax.fori_loop(0, ngrp, grp, 0)
        for b in range(2):
            pltpu.make_async_copy(
                bufs[b], acc.at[colc.at[b]], ssems[b]).wait()
        plsc.subcore_barrier()
        pltpu.sync_copy(acc.at[sl], out_hbm.at[c].at[sl])

    return spmm(T, rc3, w3)


def _deg_call(rc3, w3):
    mesh = plsc.VectorSubcoreMesh(core_axis_name="c", subcore_axis_name="s")

    @functools.partial(
        pl.kernel,
        mesh=mesh,
        out_type=jax.ShapeDtypeStruct((2, PLOW, H), _f32),
        scratch_types=[
            pltpu.VMEM((EPT,), jnp.int32),
            pltpu.VMEM((EPT,), _f32),
            pltpu.VMEM((1, CE), jnp.int32),
            pltpu.VMEM((CE, H), _f32),
            pltpu.VMEM_SHARED((NLOW, H), _f32),
        ],
    )
    def degk(rc_hbm, w_hbm, out_hbm, rc_v, w_v, colc, rows, acc):
        c = lax.axis_index("c")
        s = lax.axis_index("s")
        eslice = c * NT + s
        nrt = NLOW // NT
        sl = pl.ds(s * nrt, nrt)

        pltpu.sync_copy(rc_hbm.at[eslice], rc_v)
        pltpu.sync_copy(w_hbm.at[eslice], w_v)

        z = jnp.zeros((16,), _f32)
        for r in range(CE):
            rr = rows.at[r]
            for fg in range(H // 16):
                rr[pl.ds(fg * 16, 16)] = z
        for q in range(nrt // CE):
            pltpu.sync_copy(rows, acc.at[pl.ds(s * nrt + q * CE, CE)])
        rem = nrt - (nrt // CE) * CE
        if rem:
            pltpu.sync_copy(rows.at[pl.ds(0, rem)],
                            acc.at[pl.ds(s * nrt + nrt - rem, rem)])
        plsc.subcore_barrier()

        def chunk(j, carry):
            def sk(k, carry2):
                pv = rc_v[pl.ds(j * CE + k * 16, 16)]
                colc.at[0][pl.ds(k * 16, 16)] = pv >> 16
                wvec = w_v[pl.ds(j * CE + k * 16, 16)]
                for e in range(16):
                    wsp = _bcast_lane(wvec, e)
                    rr = rows.at[k * 16 + e]
                    for fg in range(H // 16):
                        rr[pl.ds(fg * 16, 16)] = wsp
                return carry2

            lax.fori_loop(0, CE // 16, sk, 0)
            pltpu.sync_copy(rows, acc.at[colc.at[0]], add=True)
            return carry

        lax.fori_loop(0, NCH, chunk, 0)
        plsc.subcore_barrier()
        pltpu.sync_copy(acc.at[sl], out_hbm.at[c].at[sl])

    return degk(rc3, w3)


def _w_spec(shape):
    return pl.BlockSpec(shape, lambda i: (0,) * len(shape))


_ROW = pl.BlockSpec((RB, H), lambda i: (i, 0))
_ROW1 = pl.BlockSpec((RB, 1), lambda i: (i, 0))
_PLOW = pl.BlockSpec((2, RB, H), lambda i: (0, jnp.minimum(i, LOWB - 1), 0))


def _dot(a, b):
    return jnp.dot(a, b, preferred_element_type=_f32)


def _combine(p_ref, t_ref):
    gr = (pl.program_id(0) * RB
          + lax.broadcasted_iota(jnp.int32, (RB, H), 0))
    return t_ref[...] + jnp.where(gr < NLOW, p_ref[0] + p_ref[1], 0.0)


def _tc_prep(x, deg, w1, b1, w2, b2, gw):
    def body(x_ref, deg_ref, w1_ref, b1_ref, w2_ref, b2_ref, gw_ref, out_ref):
        h = jnp.maximum(x_ref[...] * w1_ref[...] + b1_ref[...], 0.0)
        h = _dot(h, w2_ref[...]) + b2_ref[...]
        out_ref[...] = lax.rsqrt(deg_ref[...]) * _dot(h, gw_ref[...])

    return pl.pallas_call(
        body,
        grid=(GRID,),
        in_specs=[_ROW1, _ROW1, _w_spec((1, H)), _w_spec((1, H)),
                  _w_spec((H, H)), _w_spec((1, H)), _w_spec((H, H))],
        out_specs=_ROW,
        out_shape=jax.ShapeDtypeStruct((N2P, H), _f32),
    )(x, deg, w1, b1, w2, b2, gw)


def _tc_h0(p, t, deg, gb):
    def body(p_ref, t_ref, deg_ref, gb_ref, out_ref):
        sf = _combine(p_ref, t_ref)
        out_ref[...] = jnp.maximum(
            lax.rsqrt(deg_ref[...]) * sf + gb_ref[...], 0.0)

    return pl.pallas_call(
        body,
        grid=(GRID,),
        in_specs=[_PLOW, _ROW, _ROW1, _w_spec((1, H))],
        out_specs=_ROW,
        out_shape=jax.ShapeDtypeStruct((N2P, H), _f32),
    )(p, t, deg, gb)


def _tc_a(x, deg, mw1, mb1, mw2, mb2, gw1):
    def body(x_ref, deg_ref, mw1_ref, mb1_ref, mw2_ref, mb2_ref, gw1_ref,
             hn_ref, t_ref):
        x = x_ref[...]
        hn = _dot(jnp.tanh(_dot(x, mw1_ref[...]) + mb1_ref[...]),
                  mw2_ref[...]) + mb2_ref[...]
        hn_ref[...] = hn
        t_ref[...] = lax.rsqrt(deg_ref[...]) * _dot(x, gw1_ref[...])

    return pl.pallas_call(
        body,
        grid=(GRID,),
        in_specs=[_ROW, _ROW1, _w_spec((H, H)), _w_spec((1, H)),
                  _w_spec((H, H)), _w_spec((1, H)), _w_spec((H, H))],
        out_specs=[_ROW, _ROW],
        out_shape=[jax.ShapeDtypeStruct((N2P, H), _f32),
                   jax.ShapeDtypeStruct((N2P, H), _f32)],
    )(x, deg, mw1, mb1, mw2, mb2, gw1)


def _tc_b(p, t, deg, g1b, gw2):
    def body(p_ref, t_ref, deg_ref, g1b_ref, gw2_ref, out_ref):
        dinv = lax.rsqrt(deg_ref[...])
        sf = _combine(p_ref, t_ref)
        hg1 = jnp.maximum(dinv * sf + g1b_ref[...], 0.0)
        out_ref[...] = dinv * _dot(hg1, gw2_ref[...])

    return pl.pallas_call(
        body,
        grid=(GRID,),
        in_specs=[_PLOW, _ROW, _ROW1, _w_spec((1, H)), _w_spec((H, H))],
        out_specs=_ROW,
        out_shape=jax.ShapeDtypeStruct((N2P, H), _f32),
    )(p, t, deg, g1b, gw2)


def _tc_c(p, t, deg, g2b, hn, gwa, gwb, gb, hcur, ksum, a_next, wk, last):
    def body(p_ref, t_ref, deg_ref, g2b_ref, hn_ref, gwa_ref, gwb_ref,
             gb_ref, hcur_ref, *rest):
        if ksum is not None:
            ksum_ref = rest[0]
            rest = rest[1:]
        if last:
            (hnew_ref,) = rest
        else:
            xn_ref, ks_ref = rest
        hg = lax.rsqrt(deg_ref[...]) * _combine(p_ref, t_ref) + g2b_ref[...]
        hn = hn_ref[...]
        g = jax.nn.sigmoid(_dot(hg, gwa_ref[...]) + _dot(hn, gwb_ref[...])
                           + gb_ref[...])
        k = g * hg + (1.0 - g) * hn
        if last:
            hnew_ref[...] = hcur_ref[...] + (DT / 6.0) * (ksum_ref[...] + k)
        else:
            xn_ref[...] = hcur_ref[...] + a_next * k
            if ksum is None:
                ks_ref[...] = wk * k
            else:
                ks_ref[...] = ksum_ref[...] + wk * k

    in_specs = [_PLOW, _ROW, _ROW1, _w_spec((1, H)), _ROW, _w_spec((H, H)),
                _w_spec((H, H)), _w_spec((1, H)), _ROW]
    args = [p, t, deg, g2b, hn, gwa, gwb, gb, hcur]
    if ksum is not None:
        in_specs.append(_ROW)
        args.append(ksum)
    if last:
        out_specs = _ROW
        out_shape = jax.ShapeDtypeStruct((N2P, H), _f32)
    else:
        out_specs = [_ROW, _ROW]
        out_shape = [jax.ShapeDtypeStruct((N2P, H), _f32),
                     jax.ShapeDtypeStruct((N2P, H), _f32)]
    return pl.pallas_call(
        body,
        grid=(GRID,),
        in_specs=in_specs,
        out_specs=out_specs,
        out_shape=out_shape,
    )(*args)


def _tc_dec(hcur, w1, b1, w2, b2):
    def body(h_ref, w1_ref, b1_ref, w2_ref, b2_ref, out_ref):
        d = jnp.maximum(_dot(h_ref[...], w1_ref[...]) + b1_ref[...], 0.0)
        out_ref[...] = _dot(d, w2_ref[...]) + b2_ref[...]

    return pl.pallas_call(
        body,
        grid=(GRID,),
        in_specs=[_ROW, _w_spec((H, H)), _w_spec((1, H)),
                  _w_spec((H, 1)), _w_spec((1, 1))],
        out_specs=_ROW1,
        out_shape=jax.ShapeDtypeStruct((N2P, 1), _f32),
    )(hcur, w1, b1, w2, b2)


def kernel(X, edge_index, edge_weight, enc_W1, enc_b1, enc_W2, enc_b2,
           gcn_W, gcn_b, mlp_W1, mlp_b1, mlp_W2, mlp_b2,
           gc1_W, gc1_b, gc2_W, gc2_b, gate_W, gate_b,
           dec_W1, dec_b1, dec_W2, dec_b2):
    row = edge_index[0]
    col = edge_index[1]
    pad = EPAD - E
    packed = (col << 16) | row
    rc3 = jnp.pad(packed, (0, pad)).reshape(NW, EPT)
    w3 = jnp.pad(edge_weight, (0, pad)).reshape(NW, EPT)

    x2d = jnp.pad(X[:, -1, :, :].reshape(N2, 1), ((0, N2P - N2), (0, 0)))

    def r2(b):
        return b.reshape(1, -1)

    gwa = gate_W[:H]
    gwb = gate_W[H:]

    pdeg = _deg_call(rc3, w3)
    deg = jnp.concatenate(
        [1.0 + pdeg[0, :N, 0:1] + pdeg[1, :N, 0:1],
         jnp.ones((N2P - N, 1), _f32)])

    t0 = _tc_prep(x2d, deg, r2(enc_W1[0]), r2(enc_b1), enc_W2, r2(enc_b2),
                  gcn_W)
    p0 = _spmm_call(t0, rc3, w3)
    hcur = _tc_h0(p0, t0, deg, r2(gcn_b))

    for _ in range(2):
        ksum = None
        x = hcur
        for i in range(4):
            hn, t1 = _tc_a(x, deg, mlp_W1, r2(mlp_b1), mlp_W2, r2(mlp_b2),
                           gc1_W)
            p1 = _spmm_call(t1, rc3, w3)
            t2 = _tc_b(p1, t1, deg, r2(gc1_b), gc2_W)
            p2 = _spmm_call(t2, rc3, w3)
            if i < 3:
                a_next = 0.5 * DT if i < 2 else DT
                wk = 1.0 if i == 0 else 2.0
                x, ksum = _tc_c(p2, t2, deg, r2(gc2_b), hn, gwa, gwb,
                                r2(gate_b), hcur, ksum, a_next, wk,
                                last=False)
            else:
                hcur = _tc_c(p2, t2, deg, r2(gc2_b), hn, gwa, gwb,
                             r2(gate_b), hcur, ksum, 0.0, 1.0, last=True)

    out = _tc_dec(hcur, dec_W1, r2(dec_b1), dec_W2, r2(dec_b2))
    return out[:N2].reshape(2, N, 1)[:, None, :, :]


# speedup vs baseline: 1.4764x; 1.1400x over previous
import functools

import jax
import jax.numpy as jnp
from jax import lax
from jax.experimental import pallas as pl
from jax.experimental.pallas import tpu as pltpu
from jax.experimental.pallas import tpu_sc as plsc

N = 10000
N2 = 20000
N2P = 20480
NLOW = 10112
PLOW = 10240
H = 128
E = 320000
NT = 16
NW = 32
CE = 112
NCH = 90
EPT = NCH * CE
EPAD = NW * EPT
DT = 12.0 / 2.0
RB = 2048
GRID = N2P // RB
LOWB = PLOW // RB

_f32 = jnp.float32

_GDN = lax.GatherDimensionNumbers(
    offset_dims=(), collapsed_slice_dims=(0,), start_index_map=(0,))


def _bcast_lane(vec, e):
    idx = jnp.full((16, 1), e, jnp.int32)
    return lax.gather(vec, idx, _GDN, slice_sizes=(1,),
                      mode=lax.GatherScatterMode.PROMISE_IN_BOUNDS)


def _spmm_call(T, rc3, w3):
    mesh = plsc.VectorSubcoreMesh(core_axis_name="c", subcore_axis_name="s")
    nbytes = CE * H * 4

    @functools.partial(
        pl.kernel,
        mesh=mesh,
        out_type=jax.ShapeDtypeStruct((2, PLOW, H), _f32),
        scratch_types=[
            pltpu.VMEM((EPT,), jnp.int32),
            pltpu.VMEM((EPT,), _f32),
            pltpu.VMEM((2, CE), jnp.int32),
            pltpu.VMEM((2, CE), jnp.int32),
            pltpu.VMEM((CE, H), _f32),
            pltpu.VMEM((CE, H), _f32),
            pltpu.VMEM_SHARED((NLOW, H), _f32),
            pltpu.SemaphoreType.DMA,
            pltpu.SemaphoreType.DMA,
            pltpu.SemaphoreType.DMA,
            pltpu.SemaphoreType.DMA,
        ],
    )
    def spmm(t_hbm, rc_hbm, w_hbm, out_hbm,
             rc_v, w_v, rowc, colc, rows0, rows1, acc,
             gsem0, gsem1, ssem0, ssem1):
        c = lax.axis_index("c")
        s = lax.axis_index("s")
        eslice = c * NT + s
        nrt = NLOW // NT
        sl = pl.ds(s * nrt, nrt)
        bufs = (rows0, rows1)
        gsems = (gsem0, gsem1)
        ssems = (ssem0, ssem1)

        pltpu.sync_copy(rc_hbm.at[eslice], rc_v)
        pltpu.sync_copy(w_hbm.at[eslice], w_v)

        z = jnp.zeros((16,), _f32)
        for r in range(CE):
            rr = rows0.at[r]
            for fg in range(H // 16):
                rr[pl.ds(fg * 16, 16)] = z
        for q in range(nrt // CE):
            pltpu.sync_copy(rows0, acc.at[pl.ds(s * nrt + q * CE, CE)])
        rem = nrt - (nrt // CE) * CE
        if rem:
            pltpu.sync_copy(rows0.at[pl.ds(0, rem)],
                            acc.at[pl.ds(s * nrt + nrt - rem, rem)])

        mask16 = jnp.full((16,), 0xFFFF, jnp.int32)

        def unpack(j, b):
            for k in range(CE // 16):
                pv = rc_v[pl.ds(j * CE + k * 16, 16)]
                rowc.at[b][pl.ds(k * 16, 16)] = pv & mask16
                colc.at[b][pl.ds(k * 16, 16)] = pv >> 16

        for b in range(2):
            unpack(b, b)
            pltpu.async_copy(t_hbm.at[rowc.at[b]], bufs[b], gsems[b])
        plsc.subcore_barrier()

        def scale(buf, j):
            def sk(k, carry):
                wvec = w_v[pl.ds(j * CE + k * 16, 16)]
                for e in range(16):
                    wsp = _bcast_lane(wvec, e)
                    rr = buf.at[k * 16 + e]
                    for fg in range(H // 16):
                        rr[pl.ds(fg * 16, 16)] = (
                            rr[pl.ds(fg * 16, 16)] * wsp)
                return carry

            lax.fori_loop(0, CE // 16, sk, 0)

        ngrp = NCH // 2

        def grp(g, carry):
            for b in range(2):
                j = g * 2 + b
                pltpu.make_async_copy(
                    t_hbm.at[rowc.at[b]], bufs[b], gsems[b]).wait()
                scale(bufs[b], j)
                pltpu.async_copy(bufs[b], acc.at[colc.at[b]],
                                 ssems[b], add=True)

                @pl.when(j + 2 < NCH)
                def _():
                    pltpu.make_async_copy(
                        bufs[b], acc.at[colc.at[b]], ssems[b]).wait()
                    unpack(j + 2, b)
                    pltpu.async_copy(
                        t_hbm.at[rowc.at[b]], bufs[b], gsems[b])
            return carry

        lax.fori_loop(0, ngrp, grp, 0)
        for b in range(2):
            pltpu.make_async_copy(
                bufs[b], acc.at[colc.at[b]], ssems[b]).wait()
        plsc.subcore_barrier()
        pltpu.sync_copy(acc.at[sl], out_hbm.at[c].at[sl])

    return spmm(T, rc3, w3)


def _deg_call(rc3, w3):
    mesh = plsc.VectorSubcoreMesh(core_axis_name="c", subcore_axis_name="s")

    @functools.partial(
        pl.kernel,
        mesh=mesh,
        out_type=jax.ShapeDtypeStruct((2, PLOW, H), _f32),
        scratch_types=[
            pltpu.VMEM((EPT,), jnp.int32),
            pltpu.VMEM((EPT,), _f32),
            pltpu.VMEM((2, CE), jnp.int32),
            pltpu.VMEM((CE, H), _f32),
            pltpu.VMEM((CE, H), _f32),
            pltpu.VMEM_SHARED((NLOW, H), _f32),
            pltpu.SemaphoreType.DMA,
            pltpu.SemaphoreType.DMA,
        ],
    )
    def degk(rc_hbm, w_hbm, out_hbm, rc_v, w_v, colc, rows0, rows1, acc,
             ssem0, ssem1):
        c = lax.axis_index("c")
        s = lax.axis_index("s")
        eslice = c * NT + s
        nrt = NLOW // NT
        sl = pl.ds(s * nrt, nrt)
        bufs = (rows0, rows1)
        ssems = (ssem0, ssem1)

        pltpu.sync_copy(rc_hbm.at[eslice], rc_v)
        pltpu.sync_copy(w_hbm.at[eslice], w_v)

        z = jnp.zeros((16,), _f32)
        for r in range(CE):
            rr = rows0.at[r]
            for fg in range(H // 16):
                rr[pl.ds(fg * 16, 16)] = z
        for q in range(nrt // CE):
            pltpu.sync_copy(rows0, acc.at[pl.ds(s * nrt + q * CE, CE)])
        rem = nrt - (nrt // CE) * CE
        if rem:
            pltpu.sync_copy(rows0.at[pl.ds(0, rem)],
                            acc.at[pl.ds(s * nrt + nrt - rem, rem)])
        plsc.subcore_barrier()

        def fill(b, j):
            def sk(k, carry2):
                pv = rc_v[pl.ds(j * CE + k * 16, 16)]
                colc.at[b][pl.ds(k * 16, 16)] = pv >> 16
                wvec = w_v[pl.ds(j * CE + k * 16, 16)]
                for e in range(16):
                    wsp = _bcast_lane(wvec, e)
                    rr = bufs[b].at[k * 16 + e]
                    for fg in range(H // 16):
                        rr[pl.ds(fg * 16, 16)] = wsp
                return carry2

            lax.fori_loop(0, CE // 16, sk, 0)

        for b in range(2):
            fill(b, b)
            pltpu.async_copy(bufs[b], acc.at[colc.at[b]], ssems[b],
                             add=True)

        def grp(g, carry):
            for b in range(2):
                j = g * 2 + b

                @pl.when(j + 2 < NCH)
                def _():
                    pltpu.make_async_copy(
                        bufs[b], acc.at[colc.at[b]], ssems[b]).wait()
                    fill(b, j + 2)
                    pltpu.async_copy(bufs[b], acc.at[colc.at[b]],
                                     ssems[b], add=True)
            return carry

        lax.fori_loop(0, NCH // 2, grp, 0)
        for b in range(2):
            pltpu.make_async_copy(
                bufs[b], acc.at[colc.at[b]], ssems[b]).wait()
        plsc.subcore_barrier()
        pltpu.sync_copy(acc.at[sl], out_hbm.at[c].at[sl])

    return degk(rc3, w3)


def _w_spec(shape):
    return pl.BlockSpec(shape, lambda i: (0,) * len(shape))


_ROW = pl.BlockSpec((RB, H), lambda i: (i, 0))
_ROW1 = pl.BlockSpec((RB, 1), lambda i: (i, 0))
_PLOW = pl.BlockSpec((2, RB, H), lambda i: (0, jnp.minimum(i, LOWB - 1), 0))


def _dot(a, b):
    return jnp.dot(a, b, preferred_element_type=_f32)


def _combine(p_ref, t_ref):
    gr = (pl.program_id(0) * RB
          + lax.broadcasted_iota(jnp.int32, (RB, H), 0))
    return t_ref[...] + jnp.where(gr < NLOW, p_ref[0] + p_ref[1], 0.0)


def _tc_prep(x, deg, w1, b1, w2, b2, gw):
    def body(x_ref, deg_ref, w1_ref, b1_ref, w2_ref, b2_ref, gw_ref, out_ref):
        h = jnp.maximum(x_ref[...] * w1_ref[...] + b1_ref[...], 0.0)
        h = _dot(h, w2_ref[...]) + b2_ref[...]
        out_ref[...] = lax.rsqrt(deg_ref[...]) * _dot(h, gw_ref[...])

    return pl.pallas_call(
        body,
        grid=(GRID,),
        in_specs=[_ROW1, _ROW1, _w_spec((1, H)), _w_spec((1, H)),
                  _w_spec((H, H)), _w_spec((1, H)), _w_spec((H, H))],
        out_specs=_ROW,
        out_shape=jax.ShapeDtypeStruct((N2P, H), _f32),
    )(x, deg, w1, b1, w2, b2, gw)


def _tc_h0(p, t, deg, gb):
    def body(p_ref, t_ref, deg_ref, gb_ref, out_ref):
        sf = _combine(p_ref, t_ref)
        out_ref[...] = jnp.maximum(
            lax.rsqrt(deg_ref[...]) * sf + gb_ref[...], 0.0)

    return pl.pallas_call(
        body,
        grid=(GRID,),
        in_specs=[_PLOW, _ROW, _ROW1, _w_spec((1, H))],
        out_specs=_ROW,
        out_shape=jax.ShapeDtypeStruct((N2P, H), _f32),
    )(p, t, deg, gb)


def _tc_t(x, deg, gw):
    def body(x_ref, deg_ref, gw_ref, t_ref):
        t_ref[...] = lax.rsqrt(deg_ref[...]) * _dot(x_ref[...], gw_ref[...])

    return pl.pallas_call(
        body,
        grid=(GRID,),
        in_specs=[_ROW, _ROW1, _w_spec((H, H))],
        out_specs=_ROW,
        out_shape=jax.ShapeDtypeStruct((N2P, H), _f32),
    )(x, deg, gw)


def _tc_hn(x, mw1, mb1, mw2, mb2):
    def body(x_ref, mw1_ref, mb1_ref, mw2_ref, mb2_ref, hn_ref):
        x = x_ref[...]
        hn_ref[...] = _dot(jnp.tanh(_dot(x, mw1_ref[...]) + mb1_ref[...]),
                           mw2_ref[...]) + mb2_ref[...]

    return pl.pallas_call(
        body,
        grid=(GRID,),
        in_specs=[_ROW, _w_spec((H, H)), _w_spec((1, H)),
                  _w_spec((H, H)), _w_spec((1, H))],
        out_specs=_ROW,
        out_shape=jax.ShapeDtypeStruct((N2P, H), _f32),
    )(x, mw1, mb1, mw2, mb2)


def _tc_b(p, t, deg, g1b, gw2):
    def body(p_ref, t_ref, deg_ref, g1b_ref, gw2_ref, out_ref):
        dinv = lax.rsqrt(deg_ref[...])
        sf = _combine(p_ref, t_ref)
        hg1 = jnp.maximum(dinv * sf + g1b_ref[...], 0.0)
        out_ref[...] = dinv * _dot(hg1, gw2_ref[...])

    return pl.pallas_call(
        body,
        grid=(GRID,),
        in_specs=[_PLOW, _ROW, _ROW1, _w_spec((1, H)), _w_spec((H, H))],
        out_specs=_ROW,
        out_shape=jax.ShapeDtypeStruct((N2P, H), _f32),
    )(p, t, deg, g1b, gw2)


def _tc_c(p, t, deg, g2b, hn, gwa, gwb, gb, hcur, ksum, a_next, wk, last):
    def body(p_ref, t_ref, deg_ref, g2b_ref, hn_ref, gwa_ref, gwb_ref,
             gb_ref, hcur_ref, *rest):
        if ksum is not None:
            ksum_ref = rest[0]
            rest = rest[1:]
        if last:
            (hnew_ref,) = rest
        else:
            xn_ref, ks_ref = rest
        hg = lax.rsqrt(deg_ref[...]) * _combine(p_ref, t_ref) + g2b_ref[...]
        hn = hn_ref[...]
        g = jax.nn.sigmoid(_dot(hg, gwa_ref[...]) + _dot(hn, gwb_ref[...])
                           + gb_ref[...])
        k = g * hg + (1.0 - g) * hn
        if last:
            hnew_ref[...] = hcur_ref[...] + (DT / 6.0) * (ksum_ref[...] + k)
        else:
            xn_ref[...] = hcur_ref[...] + a_next * k
            if ksum is None:
                ks_ref[...] = wk * k
            else:
                ks_ref[...] = ksum_ref[...] + wk * k

    in_specs = [_PLOW, _ROW, _ROW1, _w_spec((1, H)), _ROW, _w_spec((H, H)),
                _w_spec((H, H)), _w_spec((1, H)), _ROW]
    args = [p, t, deg, g2b, hn, gwa, gwb, gb, hcur]
    if ksum is not None:
        in_specs.append(_ROW)
        args.append(ksum)
    if last:
        out_specs = _ROW
        out_shape = jax.ShapeDtypeStruct((N2P, H), _f32)
    else:
        out_specs = [_ROW, _ROW]
        out_shape = [jax.ShapeDtypeStruct((N2P, H), _f32),
                     jax.ShapeDtypeStruct((N2P, H), _f32)]
    return pl.pallas_call(
        body,
        grid=(GRID,),
        in_specs=in_specs,
        out_specs=out_specs,
        out_shape=out_shape,
    )(*args)


def _tc_dec(hcur, w1, b1, w2, b2):
    def body(h_ref, w1_ref, b1_ref, w2_ref, b2_ref, out_ref):
        d = jnp.maximum(_dot(h_ref[...], w1_ref[...]) + b1_ref[...], 0.0)
        out_ref[...] = _dot(d, w2_ref[...]) + b2_ref[...]

    return pl.pallas_call(
        body,
        grid=(GRID,),
        in_specs=[_ROW, _w_spec((H, H)), _w_spec((1, H)),
                  _w_spec((H, 1)), _w_spec((1, 1))],
        out_specs=_ROW1,
        out_shape=jax.ShapeDtypeStruct((N2P, 1), _f32),
    )(hcur, w1, b1, w2, b2)


def kernel(X, edge_index, edge_weight, enc_W1, enc_b1, enc_W2, enc_b2,
           gcn_W, gcn_b, mlp_W1, mlp_b1, mlp_W2, mlp_b2,
           gc1_W, gc1_b, gc2_W, gc2_b, gate_W, gate_b,
           dec_W1, dec_b1, dec_W2, dec_b2):
    row = edge_index[0]
    col = edge_index[1]
    pad = EPAD - E
    packed = (col << 16) | row
    rc3 = jnp.pad(packed, (0, pad)).reshape(NW, EPT)
    w3 = jnp.pad(edge_weight, (0, pad)).reshape(NW, EPT)

    x2d = jnp.pad(X[:, -1, :, :].reshape(N2, 1), ((0, N2P - N2), (0, 0)))

    def r2(b):
        return b.reshape(1, -1)

    gwa = gate_W[:H]
    gwb = gate_W[H:]

    pdeg = _deg_call(rc3, w3)
    deg = jnp.concatenate(
        [1.0 + pdeg[0, :N, 0:1] + pdeg[1, :N, 0:1],
         jnp.ones((N2P - N, 1), _f32)])

    t0 = _tc_prep(x2d, deg, r2(enc_W1[0]), r2(enc_b1), enc_W2, r2(enc_b2),
                  gcn_W)
    p0 = _spmm_call(t0, rc3, w3)
    hcur = _tc_h0(p0, t0, deg, r2(gcn_b))

    for _ in range(2):
        ksum = None
        x = hcur
        for i in range(4):
            t1 = _tc_t(x, deg, gc1_W)
            p1 = _spmm_call(t1, rc3, w3)
            hn = _tc_hn(x, mlp_W1, r2(mlp_b1), mlp_W2, r2(mlp_b2))
            t2 = _tc_b(p1, t1, deg, r2(gc1_b), gc2_W)
            p2 = _spmm_call(t2, rc3, w3)
            if i < 3:
                a_next = 0.5 * DT if i < 2 else DT
                wk = 1.0 if i == 0 else 2.0
                x, ksum = _tc_c(p2, t2, deg, r2(gc2_b), hn, gwa, gwb,
                                r2(gate_b), hcur, ksum, a_next, wk,
                                last=False)
            else:
                hcur = _tc_c(p2, t2, deg, r2(gc2_b), hn, gwa, gwb,
                             r2(gate_b), hcur, ksum, 0.0, 1.0, last=True)

    out = _tc_dec(hcur, dec_W1, r2(dec_b1), dec_W2, r2(dec_b2))
    return out[:N2].reshape(2, N, 1)[:, None, :, :]
